# Initial kernel scaffold; baseline (speedup 1.0000x reference)
import jax
import jax.numpy as jnp
from jax.experimental import pallas as pl


def kernel(indices, table, W1, b1, W2, b2):
    raise NotImplementedError("write your pallas kernel here")


# speedup vs baseline: 1.4920x; 1.4920x over previous
import functools

import jax
import jax.numpy as jnp
from jax import lax
from jax.experimental import pallas as pl
from jax.experimental.pallas import tpu as pltpu
from jax.experimental.pallas import tpu_sc as plsc

VOCAB = 1000000
DIM = 64
DP = 8

PROJ_BLOCK = 10000


def _proj_body(tab_ref, w1_ref, b1_ref, w2_ref, b2_ref, out_ref):
    h = jnp.dot(tab_ref[...], w1_ref[...],
                preferred_element_type=jnp.float32) + b1_ref[...]
    out_ref[...] = jnp.dot(h, w2_ref[...],
                           preferred_element_type=jnp.float32) + b2_ref[...]


def _project_table(table, w1p, b1p, w2p, b2p):
    grid = (VOCAB // PROJ_BLOCK,)
    return pl.pallas_call(
        _proj_body,
        grid=grid,
        in_specs=[
            pl.BlockSpec((PROJ_BLOCK, DIM), lambda i: (i, 0)),
            pl.BlockSpec((DIM, DP), lambda i: (0, 0)),
            pl.BlockSpec((1, DP), lambda i: (0, 0)),
            pl.BlockSpec((DP, DP), lambda i: (0, 0)),
            pl.BlockSpec((1, DP), lambda i: (0, 0)),
        ],
        out_specs=pl.BlockSpec((PROJ_BLOCK, DP), lambda i: (i, 0)),
        out_shape=jax.ShapeDtypeStruct((VOCAB, DP), jnp.float32),
        compiler_params=pltpu.CompilerParams(
            dimension_semantics=("arbitrary",),
        ),
    )(table, w1p, b1p, w2p, b2p)


NW = 32
IDX_W = 128
FIRE_K = 20
N_GROUPS = 10
PER_W = IDX_W * FIRE_K * N_GROUPS
GROUP_ROWS = IDX_W * FIRE_K


def _make_gather(n_tot):
    assert n_tot == NW * PER_W
    mesh = plsc.VectorSubcoreMesh(core_axis_name="c", subcore_axis_name="s")

    @functools.partial(
        pl.kernel,
        mesh=mesh,
        out_type=jax.ShapeDtypeStruct((n_tot, DP), jnp.float32),
        scratch_types=[
            pltpu.VMEM((FIRE_K * N_GROUPS, IDX_W), jnp.int32),
            pltpu.VMEM((GROUP_ROWS, DP), jnp.float32),
            pltpu.SemaphoreType.DMA,
        ],
        compiler_params=pltpu.CompilerParams(use_tc_tiling_on_sc=False),
    )
    def gather_k(idx_hbm, table_hbm, out_hbm, idx_v, rows_v, sem):
        wid = lax.axis_index("s") * 2 + lax.axis_index("c")
        base = wid * PER_W
        pltpu.sync_copy(idx_hbm.at[wid], idx_v)

        def group_body(g, _):
            copies = []
            for j in range(FIRE_K):
                cp = pltpu.async_copy(
                    table_hbm.at[idx_v.at[g * FIRE_K + j]],
                    rows_v.at[pl.ds(j * IDX_W, IDX_W)],
                    sem,
                )
                copies.append(cp)
            for cp in copies:
                cp.wait()
            pltpu.sync_copy(
                rows_v,
                out_hbm.at[pl.ds(base + g * GROUP_ROWS, GROUP_ROWS)],
            )
            return _

        lax.fori_loop(0, N_GROUPS, group_body, None)

    return gather_k


def kernel(indices, table, W1, b1, W2, b2):
    B, L = indices.shape
    n_tot = B * L

    w1p = jnp.zeros((DIM, DP), jnp.float32).at[:, :5].set(W1)
    b1p = jnp.zeros((1, DP), jnp.float32).at[0, :5].set(b1)
    w2p = jnp.zeros((DP, DP), jnp.float32).at[:5, :5].set(W2)
    b2p = jnp.zeros((1, DP), jnp.float32).at[0, :5].set(b2)

    proj = _project_table(table, w1p, b1p, w2p, b2p)

    idx = indices.reshape(NW, FIRE_K * N_GROUPS, IDX_W).astype(jnp.int32)
    gathered = _make_gather(n_tot)(idx, proj)

    return gathered[:, :5].reshape(B, L, 5)


# speedup vs baseline: 2.0251x; 1.3573x over previous
import functools

import jax
import jax.numpy as jnp
from jax import lax
from jax.experimental import pallas as pl
from jax.experimental.pallas import tpu as pltpu
from jax.experimental.pallas import tpu_sc as plsc

VOCAB = 1000000
DIM = 64
DP = 8
PACK = 16

RB = 16384
B2 = RB // PACK
GRID = 62
VP = GRID * RB


def _proj_body(tabT_ref, w1_ref, b1_ref, w2_ref, b2_ref, out_ref):
    h = lax.dot_general(tabT_ref[...], w1_ref[...],
                        dimension_numbers=(((0,), (0,)), ((), ())),
                        preferred_element_type=jnp.float32) + b1_ref[...]
    out_ref[...] = jnp.dot(h, w2_ref[...],
                           preferred_element_type=jnp.float32) + b2_ref[...]


def _project_table(tabT, w1p, b1p, w2p, b2p):
    return pl.pallas_call(
        _proj_body,
        grid=(GRID,),
        in_specs=[
            pl.BlockSpec((DIM, RB), lambda i: (0, i)),
            pl.BlockSpec((DIM, DP), lambda i: (0, 0)),
            pl.BlockSpec((1, DP), lambda i: (0, 0)),
            pl.BlockSpec((DP, DP), lambda i: (0, 0)),
            pl.BlockSpec((1, DP), lambda i: (0, 0)),
        ],
        out_specs=pl.BlockSpec((RB, DP), lambda i: (i, 0)),
        out_shape=jax.ShapeDtypeStruct((VP, DP), jnp.float32),
        compiler_params=pltpu.CompilerParams(
            dimension_semantics=("arbitrary",),
        ),
    )(tabT, w1p, b1p, w2p, b2p)


NW = 32
IDX_W = 128
FIRE_K = 20
N_GROUPS = 10
PER_W = IDX_W * FIRE_K * N_GROUPS
GROUP_ROWS = IDX_W * FIRE_K


def _make_gather(n_tot):
    assert n_tot == NW * PER_W
    mesh = plsc.VectorSubcoreMesh(core_axis_name="c", subcore_axis_name="s")

    @functools.partial(
        pl.kernel,
        mesh=mesh,
        out_type=jax.ShapeDtypeStruct((n_tot, DP), jnp.float32),
        scratch_types=[
            pltpu.VMEM((FIRE_K * N_GROUPS, IDX_W), jnp.int32),
            pltpu.VMEM((GROUP_ROWS, DP), jnp.float32),
            pltpu.SemaphoreType.DMA,
        ],
        compiler_params=pltpu.CompilerParams(use_tc_tiling_on_sc=False),
    )
    def gather_k(idx_hbm, table_hbm, out_hbm, idx_v, rows_v, sem):
        wid = lax.axis_index("s") * 2 + lax.axis_index("c")
        base = wid * PER_W
        pltpu.sync_copy(idx_hbm.at[wid], idx_v)

        def group_body(g, _):
            copies = []
            for j in range(FIRE_K):
                cp = pltpu.async_copy(
                    table_hbm.at[idx_v.at[g * FIRE_K + j]],
                    rows_v.at[pl.ds(j * IDX_W, IDX_W)],
                    sem,
                )
                copies.append(cp)
            for cp in copies:
                cp.wait()
            pltpu.sync_copy(
                rows_v,
                out_hbm.at[pl.ds(base + g * GROUP_ROWS, GROUP_ROWS)],
            )
            return _

        lax.fori_loop(0, N_GROUPS, group_body, None)

    return gather_k


def kernel(indices, table, W1, b1, W2, b2):
    B, L = indices.shape
    n_tot = B * L

    w1p = jnp.zeros((DIM, DP), jnp.float32).at[:, :5].set(W1)
    b1p = jnp.zeros((1, DP), jnp.float32).at[0, :5].set(b1)
    w2p = jnp.zeros((DP, DP), jnp.float32).at[:5, :5].set(W2)
    b2p = jnp.zeros((1, DP), jnp.float32).at[0, :5].set(b2)

    projL = _project_table(table.T, w1p, b1p, w2p, b2p)

    idx = indices.reshape(NW, FIRE_K * N_GROUPS, IDX_W).astype(jnp.int32)
    gathered = _make_gather(n_tot)(idx, projL)

    return gathered[:, :5].reshape(B, L, 5)


# speedup vs baseline: 2.8583x; 1.4115x over previous
import functools

import jax
import jax.numpy as jnp
from jax import lax
from jax.experimental import pallas as pl
from jax.experimental.pallas import tpu as pltpu
from jax.experimental.pallas import tpu_sc as plsc

VOCAB = 1000000
DIM = 64
DP = 8
PACK = 16

RB = 16384
B2 = RB // PACK
GRID = 62
VP = GRID * RB


def _proj_body(tabT_ref, w1_ref, b1_ref, w2_ref, b2_ref, out_ref):
    h = lax.dot_general(tabT_ref[...], w1_ref[...],
                        dimension_numbers=(((0,), (0,)), ((), ())),
                        preferred_element_type=jnp.float32) + b1_ref[...]
    out_ref[:, :DP] = jnp.dot(h, w2_ref[...],
                              preferred_element_type=jnp.float32) + b2_ref[...]


def _project_table(tabT, w1p, b1p, w2p, b2p):
    return pl.pallas_call(
        _proj_body,
        grid=(GRID,),
        in_specs=[
            pl.BlockSpec((DIM, RB), lambda i: (0, i)),
            pl.BlockSpec((DIM, DP), lambda i: (0, 0)),
            pl.BlockSpec((1, DP), lambda i: (0, 0)),
            pl.BlockSpec((DP, DP), lambda i: (0, 0)),
            pl.BlockSpec((1, DP), lambda i: (0, 0)),
        ],
        out_specs=pl.BlockSpec((RB, 128), lambda i: (i, 0)),
        out_shape=jax.ShapeDtypeStruct((VP, 128), jnp.float32),
        compiler_params=pltpu.CompilerParams(
            dimension_semantics=("arbitrary",),
        ),
    )(tabT, w1p, b1p, w2p, b2p)


NW = 32
IDX_W = 128
FIRE_K = 20
N_GROUPS = 10
PER_W = IDX_W * FIRE_K * N_GROUPS
GROUP_ROWS = IDX_W * FIRE_K


def _make_gather(n_tot):
    assert n_tot == NW * PER_W
    mesh = plsc.VectorSubcoreMesh(core_axis_name="c", subcore_axis_name="s")

    @functools.partial(
        pl.kernel,
        mesh=mesh,
        out_type=jax.ShapeDtypeStruct((n_tot, DP), jnp.float32),
        scratch_types=[
            pltpu.VMEM((FIRE_K * N_GROUPS, IDX_W), jnp.int32),
            pltpu.VMEM((GROUP_ROWS, DP), jnp.float32),
            pltpu.SemaphoreType.DMA,
        ],
        compiler_params=pltpu.CompilerParams(use_tc_tiling_on_sc=False),
    )
    def gather_k(idx_hbm, table_hbm, out_hbm, idx_v, rows_v, sem):
        wid = lax.axis_index("s") * 2 + lax.axis_index("c")
        base = wid * PER_W
        pltpu.sync_copy(idx_hbm.at[wid], idx_v)

        def group_body(g, _):
            copies = []
            for j in range(FIRE_K):
                cp = pltpu.async_copy(
                    table_hbm.at[idx_v.at[g * FIRE_K + j]],
                    rows_v.at[pl.ds(j * IDX_W, IDX_W)],
                    sem,
                )
                copies.append(cp)
            for cp in copies:
                cp.wait()
            pltpu.sync_copy(
                rows_v,
                out_hbm.at[pl.ds(base + g * GROUP_ROWS, GROUP_ROWS)],
            )
            return _

        lax.fori_loop(0, N_GROUPS, group_body, None)

    return gather_k


def kernel(indices, table, W1, b1, W2, b2):
    B, L = indices.shape
    n_tot = B * L

    w1p = jnp.zeros((DIM, DP), jnp.float32).at[:, :5].set(W1)
    b1p = jnp.zeros((1, DP), jnp.float32).at[0, :5].set(b1)
    w2p = jnp.zeros((DP, DP), jnp.float32).at[:5, :5].set(W2)
    b2p = jnp.zeros((1, DP), jnp.float32).at[0, :5].set(b2)

    projL = _project_table(table.T, w1p, b1p, w2p, b2p).reshape(VP * 16, DP)

    idx = (indices.astype(jnp.int32) * 16).reshape(
        NW, FIRE_K * N_GROUPS, IDX_W)
    gathered = _make_gather(n_tot)(idx, projL)

    return gathered[:, :5].reshape(B, L, 5)


# speedup vs baseline: 4.3809x; 1.5327x over previous
import functools

import jax
import jax.numpy as jnp
from jax import lax
from jax.experimental import pallas as pl
from jax.experimental.pallas import tpu as pltpu
from jax.experimental.pallas import tpu_sc as plsc

VOCAB = 1000000
DIM = 64
DP = 8
PACK = 16

RB = 16384
B2 = RB // PACK
GRID = 62
VP = GRID * RB


def _proj_body(tabT_ref, w1_ref, b1_ref, w2_ref, b2_ref, out_ref):
    h = lax.dot_general(tabT_ref[...], w1_ref[...],
                        dimension_numbers=(((0,), (0,)), ((), ())),
                        preferred_element_type=jnp.float32) + b1_ref[...]
    out_ref[:, :DP] = jnp.dot(h, w2_ref[...],
                              preferred_element_type=jnp.float32) + b2_ref[...]


def _project_table(tabT, w1p, b1p, w2p, b2p):
    return pl.pallas_call(
        _proj_body,
        grid=(GRID,),
        in_specs=[
            pl.BlockSpec((DIM, RB), lambda i: (0, i)),
            pl.BlockSpec((DIM, DP), lambda i: (0, 0)),
            pl.BlockSpec((1, DP), lambda i: (0, 0)),
            pl.BlockSpec((DP, DP), lambda i: (0, 0)),
            pl.BlockSpec((1, DP), lambda i: (0, 0)),
        ],
        out_specs=pl.BlockSpec((RB, 128), lambda i: (i, 0)),
        out_shape=jax.ShapeDtypeStruct((VP, 128), jnp.float32),
        compiler_params=pltpu.CompilerParams(
            dimension_semantics=("arbitrary",),
        ),
    )(tabT, w1p, b1p, w2p, b2p)


NW = 32
IDX_W = 128
FIRE_K = 20
N_GROUPS = 10
PER_W = IDX_W * FIRE_K * N_GROUPS
GROUP_ROWS = IDX_W * FIRE_K


def _make_gather(n_tot):
    assert n_tot == NW * PER_W
    mesh = plsc.VectorSubcoreMesh(core_axis_name="c", subcore_axis_name="s")

    @functools.partial(
        pl.kernel,
        mesh=mesh,
        out_type=jax.ShapeDtypeStruct((5, n_tot), jnp.float32),
        scratch_types=[
            pltpu.VMEM((FIRE_K, 5, IDX_W), jnp.int32),
            pltpu.VMEM((5, GROUP_ROWS), jnp.float32),
            pltpu.SemaphoreType.DMA,
        ],
        compiler_params=pltpu.CompilerParams(use_tc_tiling_on_sc=False),
    )
    def gather_k(idx_hbm, table_hbm, out_hbm, idx_v, rowsT_v, sem):
        wid = lax.axis_index("s") * 2 + lax.axis_index("c")
        base = wid * PER_W

        def group_body(g, _):
            pltpu.sync_copy(idx_hbm.at[wid, pl.ds(g * FIRE_K, FIRE_K)], idx_v)
            for c in range(5):
                copies = []
                for j in range(FIRE_K):
                    cp = pltpu.async_copy(
                        table_hbm.at[idx_v.at[j, c]],
                        rowsT_v.at[c, pl.ds(j * IDX_W, IDX_W)],
                        sem,
                    )
                    copies.append(cp)
                for cp in copies:
                    cp.wait()
            for c in range(5):
                pltpu.sync_copy(
                    rowsT_v.at[c],
                    out_hbm.at[c, pl.ds(base + g * GROUP_ROWS, GROUP_ROWS)],
                )
            return _

        lax.fori_loop(0, N_GROUPS, group_body, None)

    return gather_k


def kernel(indices, table, W1, b1, W2, b2):
    B, L = indices.shape
    n_tot = B * L

    w1p = jnp.zeros((DIM, DP), jnp.float32).at[:, :5].set(W1)
    b1p = jnp.zeros((1, DP), jnp.float32).at[0, :5].set(b1)
    w2p = jnp.zeros((DP, DP), jnp.float32).at[:5, :5].set(W2)
    b2p = jnp.zeros((1, DP), jnp.float32).at[0, :5].set(b2)

    projL = _project_table(table.T, w1p, b1p, w2p, b2p).reshape(VP * 128)

    idxT = (indices.astype(jnp.int32) * 128).T.reshape(
        NW, N_GROUPS * FIRE_K, 1, IDX_W)
    idx5 = idxT + jnp.arange(5, dtype=jnp.int32).reshape(1, 1, 5, 1)
    idx5 = idx5.reshape(NW, N_GROUPS * FIRE_K, 5, IDX_W)
    out5 = _make_gather(n_tot)(idx5, projL)

    return out5.reshape(5, L, B).transpose(2, 1, 0)


# speedup vs baseline: 4.6454x; 1.0604x over previous
import functools

import jax
import jax.numpy as jnp
from jax import lax
from jax.experimental import pallas as pl
from jax.experimental.pallas import tpu as pltpu
from jax.experimental.pallas import tpu_sc as plsc

VOCAB = 1000000
DIM = 64
DP = 8
PACK = 16

RB = 16384
B2 = RB // PACK
GRID = 62
VP = GRID * RB


def _proj_body(tabT_ref, w1_ref, b1_ref, w2_ref, b2_ref, out_ref):
    h = lax.dot_general(tabT_ref[...], w1_ref[...],
                        dimension_numbers=(((0,), (0,)), ((), ())),
                        preferred_element_type=jnp.float32) + b1_ref[...]
    out_ref[:, :DP] = jnp.dot(h, w2_ref[...],
                              preferred_element_type=jnp.float32) + b2_ref[...]


def _project_table(tabT, w1p, b1p, w2p, b2p):
    return pl.pallas_call(
        _proj_body,
        grid=(GRID,),
        in_specs=[
            pl.BlockSpec((DIM, RB), lambda i: (0, i)),
            pl.BlockSpec((DIM, DP), lambda i: (0, 0)),
            pl.BlockSpec((1, DP), lambda i: (0, 0)),
            pl.BlockSpec((DP, DP), lambda i: (0, 0)),
            pl.BlockSpec((1, DP), lambda i: (0, 0)),
        ],
        out_specs=pl.BlockSpec((RB, 128), lambda i: (i, 0)),
        out_shape=jax.ShapeDtypeStruct((VP, 128), jnp.float32),
        compiler_params=pltpu.CompilerParams(
            dimension_semantics=("arbitrary",),
        ),
    )(tabT, w1p, b1p, w2p, b2p)


NW = 32
IDX_W = 128
FIRE_K = 20
N_GROUPS = 10
PER_W = IDX_W * FIRE_K * N_GROUPS
GROUP_ROWS = IDX_W * FIRE_K


def _make_gather(n_tot):
    assert n_tot == NW * PER_W
    mesh = plsc.VectorSubcoreMesh(core_axis_name="c", subcore_axis_name="s")

    @functools.partial(
        pl.kernel,
        mesh=mesh,
        out_type=jax.ShapeDtypeStruct((5, n_tot), jnp.float32),
        scratch_types=[
            pltpu.VMEM((FIRE_K, 5, IDX_W), jnp.int32),
            pltpu.VMEM((5, GROUP_ROWS), jnp.float32),
            pltpu.SemaphoreType.DMA,
        ],
        compiler_params=pltpu.CompilerParams(use_tc_tiling_on_sc=False),
    )
    def gather_k(idx_hbm, table_hbm, out_hbm, idx_v, rowsT_v, sem):
        wid = lax.axis_index("s") * 2 + lax.axis_index("c")
        base = wid * PER_W

        def group_body(g, _):
            pltpu.sync_copy(idx_hbm.at[wid, pl.ds(g * FIRE_K, FIRE_K)], idx_v)
            copies = []
            for c in range(5):
                for j in range(FIRE_K):
                    cp = pltpu.async_copy(
                        table_hbm.at[idx_v.at[j, c]],
                        rowsT_v.at[c, pl.ds(j * IDX_W, IDX_W)],
                        sem,
                    )
                    copies.append(cp)
            for cp in copies:
                cp.wait()
            for c in range(5):
                pltpu.sync_copy(
                    rowsT_v.at[c],
                    out_hbm.at[c, pl.ds(base + g * GROUP_ROWS, GROUP_ROWS)],
                )
            return _

        lax.fori_loop(0, N_GROUPS, group_body, None)

    return gather_k


def kernel(indices, table, W1, b1, W2, b2):
    B, L = indices.shape
    n_tot = B * L

    w1p = jnp.zeros((DIM, DP), jnp.float32).at[:, :5].set(W1)
    b1p = jnp.zeros((1, DP), jnp.float32).at[0, :5].set(b1)
    w2p = jnp.zeros((DP, DP), jnp.float32).at[:5, :5].set(W2)
    b2p = jnp.zeros((1, DP), jnp.float32).at[0, :5].set(b2)

    projL = _project_table(table.T, w1p, b1p, w2p, b2p).reshape(VP * 128)

    idxT = (indices.astype(jnp.int32) * 128).T.reshape(
        NW, N_GROUPS * FIRE_K, 1, IDX_W)
    idx5 = idxT + jnp.arange(5, dtype=jnp.int32).reshape(1, 1, 5, 1)
    idx5 = idx5.reshape(NW, N_GROUPS * FIRE_K, 5, IDX_W)
    out5 = _make_gather(n_tot)(idx5, projL)

    return out5.reshape(5, L, B).transpose(2, 1, 0)


# speedup vs baseline: 4.7290x; 1.0180x over previous
import functools

import jax
import jax.numpy as jnp
from jax import lax
from jax.experimental import pallas as pl
from jax.experimental.pallas import tpu as pltpu
from jax.experimental.pallas import tpu_sc as plsc

VOCAB = 1000000
DIM = 64
DP = 8
PACK = 16

RB = 16384
B2 = RB // PACK
GRID = 62
VP = GRID * RB


def _proj_body(tabT_ref, w1_ref, b1_ref, w2_ref, b2_ref, out_ref):
    h = lax.dot_general(tabT_ref[...], w1_ref[...],
                        dimension_numbers=(((0,), (0,)), ((), ())),
                        preferred_element_type=jnp.float32) + b1_ref[...]
    out_ref[:, :DP] = jnp.dot(h, w2_ref[...],
                              preferred_element_type=jnp.float32) + b2_ref[...]


def _project_table(tabT, w1p, b1p, w2p, b2p):
    return pl.pallas_call(
        _proj_body,
        grid=(GRID,),
        in_specs=[
            pl.BlockSpec((DIM, RB), lambda i: (0, i)),
            pl.BlockSpec((DIM, DP), lambda i: (0, 0)),
            pl.BlockSpec((1, DP), lambda i: (0, 0)),
            pl.BlockSpec((DP, DP), lambda i: (0, 0)),
            pl.BlockSpec((1, DP), lambda i: (0, 0)),
        ],
        out_specs=pl.BlockSpec((RB, 128), lambda i: (i, 0)),
        out_shape=jax.ShapeDtypeStruct((VP, 128), jnp.float32),
        compiler_params=pltpu.CompilerParams(
            dimension_semantics=("arbitrary",),
        ),
    )(tabT, w1p, b1p, w2p, b2p)


NW = 32
IDX_W = 128
FIRE_K = 40
N_GROUPS = 5
PER_W = IDX_W * FIRE_K * N_GROUPS
GROUP_ROWS = IDX_W * FIRE_K


def _make_gather(n_tot):
    assert n_tot == NW * PER_W
    mesh = plsc.VectorSubcoreMesh(core_axis_name="c", subcore_axis_name="s")

    @functools.partial(
        pl.kernel,
        mesh=mesh,
        out_type=jax.ShapeDtypeStruct((5, n_tot), jnp.float32),
        scratch_types=[
            pltpu.VMEM((FIRE_K, 5, IDX_W), jnp.int32),
            pltpu.VMEM((5, GROUP_ROWS), jnp.float32),
            pltpu.SemaphoreType.DMA,
        ],
        compiler_params=pltpu.CompilerParams(use_tc_tiling_on_sc=False),
    )
    def gather_k(idx_hbm, table_hbm, out_hbm, idx_v, rowsT_v, sem):
        wid = lax.axis_index("s") * 2 + lax.axis_index("c")
        base = wid * PER_W

        def group_body(g, _):
            pltpu.sync_copy(idx_hbm.at[wid, pl.ds(g * FIRE_K, FIRE_K)], idx_v)
            copies = []
            for c in range(5):
                for j in range(FIRE_K):
                    cp = pltpu.async_copy(
                        table_hbm.at[idx_v.at[j, c]],
                        rowsT_v.at[c, pl.ds(j * IDX_W, IDX_W)],
                        sem,
                    )
                    copies.append(cp)
            for cp in copies:
                cp.wait()
            for c in range(5):
                pltpu.sync_copy(
                    rowsT_v.at[c],
                    out_hbm.at[c, pl.ds(base + g * GROUP_ROWS, GROUP_ROWS)],
                )
            return _

        lax.fori_loop(0, N_GROUPS, group_body, None)

    return gather_k


def kernel(indices, table, W1, b1, W2, b2):
    B, L = indices.shape
    n_tot = B * L

    w1p = jnp.zeros((DIM, DP), jnp.float32).at[:, :5].set(W1)
    b1p = jnp.zeros((1, DP), jnp.float32).at[0, :5].set(b1)
    w2p = jnp.zeros((DP, DP), jnp.float32).at[:5, :5].set(W2)
    b2p = jnp.zeros((1, DP), jnp.float32).at[0, :5].set(b2)

    projL = _project_table(table.T, w1p, b1p, w2p, b2p).reshape(VP * 128)

    idxT = (indices.astype(jnp.int32) * 128).T.reshape(
        NW, N_GROUPS * FIRE_K, 1, IDX_W)
    idx5 = idxT + jnp.arange(5, dtype=jnp.int32).reshape(1, 1, 5, 1)
    idx5 = idx5.reshape(NW, N_GROUPS * FIRE_K, 5, IDX_W)
    out5 = _make_gather(n_tot)(idx5, projL)

    return out5.reshape(5, L, B).transpose(2, 1, 0)


# speedup vs baseline: 4.7904x; 1.0130x over previous
import functools

import jax
import jax.numpy as jnp
from jax import lax
from jax.experimental import pallas as pl
from jax.experimental.pallas import tpu as pltpu
from jax.experimental.pallas import tpu_sc as plsc

VOCAB = 1000000
DIM = 64
DP = 8
PACK = 16

RB = 32768
B2 = RB // PACK
GRID = 31
VP = GRID * RB


def _proj_body(tabT_ref, w1_ref, b1_ref, w2_ref, b2_ref, out_ref):
    h = lax.dot_general(tabT_ref[...], w1_ref[...],
                        dimension_numbers=(((0,), (0,)), ((), ())),
                        preferred_element_type=jnp.float32) + b1_ref[...]
    out_ref[:, :DP] = jnp.dot(h, w2_ref[...],
                              preferred_element_type=jnp.float32) + b2_ref[...]


def _project_table(tabT, w1p, b1p, w2p, b2p):
    return pl.pallas_call(
        _proj_body,
        grid=(GRID,),
        in_specs=[
            pl.BlockSpec((DIM, RB), lambda i: (0, i)),
            pl.BlockSpec((DIM, DP), lambda i: (0, 0)),
            pl.BlockSpec((1, DP), lambda i: (0, 0)),
            pl.BlockSpec((DP, DP), lambda i: (0, 0)),
            pl.BlockSpec((1, DP), lambda i: (0, 0)),
        ],
        out_specs=pl.BlockSpec((RB, 128), lambda i: (i, 0)),
        out_shape=jax.ShapeDtypeStruct((VP, 128), jnp.float32),
        compiler_params=pltpu.CompilerParams(
            dimension_semantics=("arbitrary",),
        ),
    )(tabT, w1p, b1p, w2p, b2p)


NW = 32
IDX_W = 128
FIRE_K = 40
N_GROUPS = 5
PER_W = IDX_W * FIRE_K * N_GROUPS
GROUP_ROWS = IDX_W * FIRE_K


def _make_gather(n_tot):
    assert n_tot == NW * PER_W
    mesh = plsc.VectorSubcoreMesh(core_axis_name="c", subcore_axis_name="s")

    @functools.partial(
        pl.kernel,
        mesh=mesh,
        out_type=jax.ShapeDtypeStruct((5, n_tot), jnp.float32),
        scratch_types=[
            pltpu.VMEM((FIRE_K, 5, IDX_W), jnp.int32),
            pltpu.VMEM((5, GROUP_ROWS), jnp.float32),
            pltpu.SemaphoreType.DMA,
        ],
        compiler_params=pltpu.CompilerParams(use_tc_tiling_on_sc=False),
    )
    def gather_k(idx_hbm, table_hbm, out_hbm, idx_v, rowsT_v, sem):
        wid = lax.axis_index("s") * 2 + lax.axis_index("c")
        base = wid * PER_W

        def group_body(g, _):
            pltpu.sync_copy(idx_hbm.at[wid, pl.ds(g * FIRE_K, FIRE_K)], idx_v)
            copies = []
            for c in range(5):
                for j in range(FIRE_K):
                    cp = pltpu.async_copy(
                        table_hbm.at[idx_v.at[j, c]],
                        rowsT_v.at[c, pl.ds(j * IDX_W, IDX_W)],
                        sem,
                    )
                    copies.append(cp)
            for cp in copies:
                cp.wait()
            for c in range(5):
                pltpu.sync_copy(
                    rowsT_v.at[c],
                    out_hbm.at[c, pl.ds(base + g * GROUP_ROWS, GROUP_ROWS)],
                )
            return _

        lax.fori_loop(0, N_GROUPS, group_body, None)

    return gather_k


def kernel(indices, table, W1, b1, W2, b2):
    B, L = indices.shape
    n_tot = B * L

    w1p = jnp.zeros((DIM, DP), jnp.float32).at[:, :5].set(W1)
    b1p = jnp.zeros((1, DP), jnp.float32).at[0, :5].set(b1)
    w2p = jnp.zeros((DP, DP), jnp.float32).at[:5, :5].set(W2)
    b2p = jnp.zeros((1, DP), jnp.float32).at[0, :5].set(b2)

    projL = _project_table(table.T, w1p, b1p, w2p, b2p).reshape(VP * 128)

    idxT = (indices.astype(jnp.int32) * 128).T.reshape(
        NW, N_GROUPS * FIRE_K, 1, IDX_W)
    idx5 = idxT + jnp.arange(5, dtype=jnp.int32).reshape(1, 1, 5, 1)
    idx5 = idx5.reshape(NW, N_GROUPS * FIRE_K, 5, IDX_W)
    out5 = _make_gather(n_tot)(idx5, projL)

    return out5.reshape(5, L, B).transpose(2, 1, 0)


# speedup vs baseline: 5.0902x; 1.0626x over previous
import functools

import jax
import jax.numpy as jnp
from jax import lax
from jax.experimental import pallas as pl
from jax.experimental.pallas import tpu as pltpu
from jax.experimental.pallas import tpu_sc as plsc

VOCAB = 1000000
DIM = 64
DP = 8
PACK = 16

RB = 32768
B2 = RB // PACK
GRID = 31
VP = GRID * RB


def _proj_body(tabT_ref, w1_ref, b1_ref, w2_ref, b2_ref, out_ref):
    h = lax.dot_general(tabT_ref[...], w1_ref[...],
                        dimension_numbers=(((0,), (0,)), ((), ())),
                        preferred_element_type=jnp.float32) + b1_ref[...]
    out_ref[:, :DP] = jnp.dot(h, w2_ref[...],
                              preferred_element_type=jnp.float32) + b2_ref[...]


def _project_table(tabT, w1p, b1p, w2p, b2p):
    return pl.pallas_call(
        _proj_body,
        grid=(GRID,),
        in_specs=[
            pl.BlockSpec((DIM, RB), lambda i: (0, i)),
            pl.BlockSpec((DIM, DP), lambda i: (0, 0)),
            pl.BlockSpec((1, DP), lambda i: (0, 0)),
            pl.BlockSpec((DP, DP), lambda i: (0, 0)),
            pl.BlockSpec((1, DP), lambda i: (0, 0)),
        ],
        out_specs=pl.BlockSpec((RB, 128), lambda i: (i, 0)),
        out_shape=jax.ShapeDtypeStruct((VP, 128), jnp.float32),
        compiler_params=pltpu.CompilerParams(
            dimension_semantics=("arbitrary",),
        ),
    )(tabT, w1p, b1p, w2p, b2p)


NW = 32
IDX_W = 128
FIRE_K = 40
N_GROUPS = 5
PER_W = IDX_W * FIRE_K * N_GROUPS
GROUP_ROWS = IDX_W * FIRE_K


def _make_gather(n_tot):
    assert n_tot == NW * PER_W
    mesh = plsc.VectorSubcoreMesh(core_axis_name="c", subcore_axis_name="s")

    @functools.partial(
        pl.kernel,
        mesh=mesh,
        out_type=jax.ShapeDtypeStruct((5, n_tot), jnp.float32),
        scratch_types=[
            pltpu.VMEM((FIRE_K, IDX_W), jnp.int32),
            pltpu.VMEM((FIRE_K, 5, IDX_W), jnp.int32),
            pltpu.VMEM((5, GROUP_ROWS), jnp.float32),
            pltpu.SemaphoreType.DMA,
        ],
        compiler_params=pltpu.CompilerParams(use_tc_tiling_on_sc=False),
    )
    def gather_k(idx_hbm, table_hbm, out_hbm, idxb_v, idx_v, rowsT_v, sem):
        wid = lax.axis_index("s") * 2 + lax.axis_index("c")
        base = wid * PER_W

        def group_body(g, _):
            pltpu.sync_copy(idx_hbm.at[wid, pl.ds(g * FIRE_K, FIRE_K)], idxb_v)

            def exp_body(j, _):
                for k in range(IDX_W // 16):
                    b16 = idxb_v[j, pl.ds(k * 16, 16)]
                    for c in range(5):
                        idx_v[j, c, pl.ds(k * 16, 16)] = b16 + c
                return _

            lax.fori_loop(0, FIRE_K, exp_body, None)
            copies = []
            for c in range(5):
                for j in range(FIRE_K):
                    cp = pltpu.async_copy(
                        table_hbm.at[idx_v.at[j, c]],
                        rowsT_v.at[c, pl.ds(j * IDX_W, IDX_W)],
                        sem,
                    )
                    copies.append(cp)
            for cp in copies:
                cp.wait()
            for c in range(5):
                pltpu.sync_copy(
                    rowsT_v.at[c],
                    out_hbm.at[c, pl.ds(base + g * GROUP_ROWS, GROUP_ROWS)],
                )
            return _

        lax.fori_loop(0, N_GROUPS, group_body, None)

    return gather_k


def kernel(indices, table, W1, b1, W2, b2):
    B, L = indices.shape
    n_tot = B * L

    w1p = jnp.zeros((DIM, DP), jnp.float32).at[:, :5].set(W1)
    b1p = jnp.zeros((1, DP), jnp.float32).at[0, :5].set(b1)
    w2p = jnp.zeros((DP, DP), jnp.float32).at[:5, :5].set(W2)
    b2p = jnp.zeros((1, DP), jnp.float32).at[0, :5].set(b2)

    projL = _project_table(table.T, w1p, b1p, w2p, b2p).reshape(VP * 128)

    idxT = (indices.astype(jnp.int32) * 128).T.reshape(
        NW, N_GROUPS * FIRE_K, IDX_W)
    out5 = _make_gather(n_tot)(idxT, projL)

    return out5.reshape(5, L, B).transpose(2, 1, 0)


# speedup vs baseline: 5.1043x; 1.0028x over previous
import functools

import jax
import jax.numpy as jnp
from jax import lax
from jax.experimental import pallas as pl
from jax.experimental.pallas import tpu as pltpu
from jax.experimental.pallas import tpu_sc as plsc

VOCAB = 1000000
DIM = 64
DP = 8
PACK = 16

RB = 32768
B2 = RB // PACK
GRID = 31
VP = GRID * RB


def _proj_body(tabT_ref, w1_ref, b1_ref, w2_ref, b2_ref, out_ref):
    h = lax.dot_general(tabT_ref[...], w1_ref[...],
                        dimension_numbers=(((0,), (0,)), ((), ())),
                        preferred_element_type=jnp.float32) + b1_ref[...]
    out_ref[:, :DP] = jnp.dot(h, w2_ref[...],
                              preferred_element_type=jnp.float32) + b2_ref[...]


def _project_table(tabT, w1p, b1p, w2p, b2p):
    return pl.pallas_call(
        _proj_body,
        grid=(GRID,),
        in_specs=[
            pl.BlockSpec((DIM, RB), lambda i: (0, i)),
            pl.BlockSpec((DIM, DP), lambda i: (0, 0)),
            pl.BlockSpec((1, DP), lambda i: (0, 0)),
            pl.BlockSpec((DP, DP), lambda i: (0, 0)),
            pl.BlockSpec((1, DP), lambda i: (0, 0)),
        ],
        out_specs=pl.BlockSpec((RB, 128), lambda i: (i, 0)),
        out_shape=jax.ShapeDtypeStruct((VP, 128), jnp.float32),
        compiler_params=pltpu.CompilerParams(
            dimension_semantics=("arbitrary",),
        ),
    )(tabT, w1p, b1p, w2p, b2p)


NW = 32
IDX_W = 128
FIRE_K = 50
N_GROUPS = 4
PER_W = IDX_W * FIRE_K * N_GROUPS
GROUP_ROWS = IDX_W * FIRE_K


def _make_gather(n_tot):
    assert n_tot == NW * PER_W
    mesh = plsc.VectorSubcoreMesh(core_axis_name="c", subcore_axis_name="s")

    @functools.partial(
        pl.kernel,
        mesh=mesh,
        out_type=jax.ShapeDtypeStruct((5, n_tot), jnp.float32),
        scratch_types=[
            pltpu.VMEM((FIRE_K, IDX_W), jnp.int32),
            pltpu.VMEM((FIRE_K, 5, IDX_W), jnp.int32),
            pltpu.VMEM((5, GROUP_ROWS), jnp.float32),
            pltpu.SemaphoreType.DMA,
        ],
        compiler_params=pltpu.CompilerParams(use_tc_tiling_on_sc=False),
    )
    def gather_k(idx_hbm, table_hbm, out_hbm, idxb_v, idx_v, rowsT_v, sem):
        wid = lax.axis_index("s") * 2 + lax.axis_index("c")
        base = wid * PER_W

        def group_body(g, _):
            pltpu.sync_copy(idx_hbm.at[wid, pl.ds(g * FIRE_K, FIRE_K)], idxb_v)

            def exp_body(j, _):
                for k in range(IDX_W // 16):
                    b16 = idxb_v[j, pl.ds(k * 16, 16)]
                    for c in range(5):
                        idx_v[j, c, pl.ds(k * 16, 16)] = b16 + c
                return _

            lax.fori_loop(0, FIRE_K, exp_body, None)
            copies = []
            for c in range(5):
                for j in range(FIRE_K):
                    cp = pltpu.async_copy(
                        table_hbm.at[idx_v.at[j, c]],
                        rowsT_v.at[c, pl.ds(j * IDX_W, IDX_W)],
                        sem,
                    )
                    copies.append(cp)
            for cp in copies:
                cp.wait()
            for c in range(5):
                pltpu.sync_copy(
                    rowsT_v.at[c],
                    out_hbm.at[c, pl.ds(base + g * GROUP_ROWS, GROUP_ROWS)],
                )
            return _

        lax.fori_loop(0, N_GROUPS, group_body, None)

    return gather_k


def kernel(indices, table, W1, b1, W2, b2):
    B, L = indices.shape
    n_tot = B * L

    w1p = jnp.zeros((DIM, DP), jnp.float32).at[:, :5].set(W1)
    b1p = jnp.zeros((1, DP), jnp.float32).at[0, :5].set(b1)
    w2p = jnp.zeros((DP, DP), jnp.float32).at[:5, :5].set(W2)
    b2p = jnp.zeros((1, DP), jnp.float32).at[0, :5].set(b2)

    projL = _project_table(table.T, w1p, b1p, w2p, b2p).reshape(VP * 128)

    idxT = (indices.astype(jnp.int32) * 128).T.reshape(
        NW, N_GROUPS * FIRE_K, IDX_W)
    out5 = _make_gather(n_tot)(idxT, projL)

    return out5.reshape(5, L, B).transpose(2, 1, 0)
